# Initial kernel scaffold; baseline (speedup 1.0000x reference)
import jax
import jax.numpy as jnp
from jax.experimental import pallas as pl


def kernel(x, a, layout, kernel1, attn_s1, attn_n1, bias1, Wl, bl, kernel2, attn_s2, attn_n2, bias2):
    raise NotImplementedError("write your pallas kernel here")


# speedup vs baseline: 1.3248x; 1.3248x over previous
import functools

import jax
import jax.numpy as jnp
from jax.experimental import pallas as pl

BN = 256


def _leaky_relu(v):
    return jnp.where(v >= 0, v, 0.2 * v)


def _k1_body(x_ref, k1_ref, ast_ref, ant_ref, lay_ref, wl_ref, bl_ref,
             h1_ref, s1_ref, t1_ref, lemb_ref):
    h1 = jnp.dot(x_ref[...], k1_ref[...], preferred_element_type=jnp.float32)
    h1_ref[...] = h1
    s1_ref[...] = jax.lax.dot_general(
        h1, ast_ref[...], (((1,), (1,)), ((), ())),
        preferred_element_type=jnp.float32)
    t1_ref[...] = jax.lax.dot_general(
        ant_ref[...], h1, (((1,), (1,)), ((), ())),
        preferred_element_type=jnp.float32)
    lemb = jnp.dot(lay_ref[...], wl_ref[...],
                   preferred_element_type=jnp.float32) + bl_ref[...]
    lemb_ref[...] = jnp.maximum(lemb, 0.0)


def _k2_body(a_ref, s1_ref, t1_ref, h1_ref, lemb_ref, b1_ref, w2_ref,
             h2_ref, *, heads, chan):
    mask = a_ref[...] > 0.0
    s1b = s1_ref[...]
    t1r = t1_ref[...]
    h1f = h1_ref[...]
    outs = []
    for h in range(heads):
        logit = _leaky_relu(s1b[:, h:h + 1] + t1r[h:h + 1, :])
        logit = jnp.where(mask, logit, jnp.float32(-1e9))
        rmax = jnp.max(logit, axis=1, keepdims=True)
        p = jnp.exp(logit - rmax)
        denom = jnp.sum(p, axis=1, keepdims=True)
        alpha = p / denom
        outs.append(jnp.dot(alpha, h1f[:, h * chan:(h + 1) * chan],
                            preferred_element_type=jnp.float32))
    x1 = jnp.concatenate(outs, axis=1) + b1_ref[...]
    x1 = jnp.where(x1 > 0, x1, jnp.exp(x1) - 1.0)
    x1g = x1 + lemb_ref[...]
    h2_ref[...] = jnp.dot(x1g, w2_ref[...], preferred_element_type=jnp.float32)


def _k3_body(a_ref, h2_ref, as2_ref, an2_ref, b2_ref, out_ref, *, bn):
    i = pl.program_id(0)
    h2f = h2_ref[...]
    h2b = h2_ref[pl.ds(i * bn, bn), :]
    s2 = jax.lax.dot_general(h2b, as2_ref[...], (((1,), (1,)), ((), ())),
                             preferred_element_type=jnp.float32)
    t2 = jax.lax.dot_general(an2_ref[...], h2f, (((1,), (1,)), ((), ())),
                             preferred_element_type=jnp.float32)
    logit = _leaky_relu(s2 + t2)
    logit = jnp.where(a_ref[...] > 0.0, logit, jnp.float32(-1e9))
    rmax = jnp.max(logit, axis=1, keepdims=True)
    p = jnp.exp(logit - rmax)
    alpha = p / jnp.sum(p, axis=1, keepdims=True)
    h2agg = jnp.dot(alpha, h2f, preferred_element_type=jnp.float32)
    h2agg = h2agg + b2_ref[...]
    cmax = jnp.max(h2agg, axis=1, keepdims=True)
    e = jnp.exp(h2agg - cmax)
    out_ref[...] = e / jnp.sum(e, axis=1, keepdims=True)


@jax.jit
def kernel(x, a, layout, kernel1, attn_s1, attn_n1, bias1, Wl, bl,
           kernel2, attn_s2, attn_n2, bias2):
    N, F = x.shape
    H, C = attn_s1.shape
    NC = attn_s2.shape[1]
    HC = H * C

    k1m = kernel1.reshape(F, HC)
    w2 = kernel2.reshape(HC, NC)
    eye = jnp.eye(H, dtype=x.dtype)
    ast = (eye[:, :, None] * attn_s1[None, :, :]).reshape(H, HC)
    ant = (eye[:, :, None] * attn_n1[None, :, :]).reshape(H, HC)
    DL = layout.shape[1]
    DLP = 8
    lay = jnp.pad(layout, ((0, 0), (0, DLP - DL)))
    wlp = jnp.pad(Wl, ((0, DLP - DL), (0, 0)))

    h1, s1, t1, lemb = pl.pallas_call(
        _k1_body,
        out_shape=(
            jax.ShapeDtypeStruct((N, HC), jnp.float32),
            jax.ShapeDtypeStruct((N, H), jnp.float32),
            jax.ShapeDtypeStruct((H, N), jnp.float32),
            jax.ShapeDtypeStruct((N, HC), jnp.float32),
        ),
    )(x, k1m, ast, ant, lay, wlp, bl.reshape(1, HC))

    nblk = N // BN
    h2 = pl.pallas_call(
        functools.partial(_k2_body, heads=H, chan=C),
        grid=(nblk,),
        in_specs=[
            pl.BlockSpec((BN, N), lambda i: (i, 0)),
            pl.BlockSpec((BN, H), lambda i: (i, 0)),
            pl.BlockSpec((H, N), lambda i: (0, 0)),
            pl.BlockSpec((N, HC), lambda i: (0, 0)),
            pl.BlockSpec((BN, HC), lambda i: (i, 0)),
            pl.BlockSpec((1, HC), lambda i: (0, 0)),
            pl.BlockSpec((HC, NC), lambda i: (0, 0)),
        ],
        out_specs=pl.BlockSpec((BN, NC), lambda i: (i, 0)),
        out_shape=jax.ShapeDtypeStruct((N, NC), jnp.float32),
    )(a, s1, t1, h1, lemb, bias1.reshape(1, HC), w2)

    out = pl.pallas_call(
        functools.partial(_k3_body, bn=BN),
        grid=(nblk,),
        in_specs=[
            pl.BlockSpec((BN, N), lambda i: (i, 0)),
            pl.BlockSpec((N, NC), lambda i: (0, 0)),
            pl.BlockSpec((1, NC), lambda i: (0, 0)),
            pl.BlockSpec((1, NC), lambda i: (0, 0)),
            pl.BlockSpec((1, NC), lambda i: (0, 0)),
        ],
        out_specs=pl.BlockSpec((BN, NC), lambda i: (i, 0)),
        out_shape=jax.ShapeDtypeStruct((N, NC), jnp.float32),
    )(a, h2, attn_s2, attn_n2, bias2.reshape(1, NC))
    return out


# speedup vs baseline: 1.3843x; 1.0449x over previous
import functools

import jax
import jax.numpy as jnp
from jax.experimental import pallas as pl

BN = 256


def _leaky_relu(v):
    return jnp.where(v >= 0, v, 0.2 * v)


def _k1_body(x_ref, k1_ref, ast_ref, ant_ref, lay_ref, wl_ref, bl_ref,
             h1_ref, s1_ref, t1_ref, lemb_ref):
    h1 = jnp.dot(x_ref[...], k1_ref[...], preferred_element_type=jnp.float32)
    h1_ref[...] = h1
    s1_ref[...] = jax.lax.dot_general(
        h1, ast_ref[...], (((1,), (1,)), ((), ())),
        preferred_element_type=jnp.float32)
    t1_ref[...] = jax.lax.dot_general(
        ant_ref[...], h1, (((1,), (1,)), ((), ())),
        preferred_element_type=jnp.float32)
    lemb = jnp.dot(lay_ref[...], wl_ref[...],
                   preferred_element_type=jnp.float32) + bl_ref[...]
    lemb_ref[...] = jnp.maximum(lemb, 0.0)


def _k2_body(a_ref, s1_ref, t1_ref, h1_ref, lemb_ref, b1_ref, w2_ref,
             h2_ref, *, heads, chan):
    abias = jnp.where(a_ref[...] > 0.0, 0.0, jnp.float32(-1e9))
    s1b = s1_ref[...]
    t1r = t1_ref[...]
    h1f = h1_ref[...]
    outs = []
    for h in range(heads):
        logit = _leaky_relu(s1b[:, h:h + 1] + t1r[h:h + 1, :]) + abias
        rmax = jnp.max(logit, axis=1, keepdims=True)
        p = jnp.exp(logit - rmax)
        denom = jnp.sum(p, axis=1, keepdims=True)
        outs.append(jnp.dot(p, h1f[:, h * chan:(h + 1) * chan],
                            preferred_element_type=jnp.float32) / denom)
    x1 = jnp.concatenate(outs, axis=1) + b1_ref[...]
    x1 = jnp.where(x1 > 0, x1, jnp.exp(x1) - 1.0)
    x1g = x1 + lemb_ref[...]
    h2_ref[...] = jnp.dot(x1g, w2_ref[...], preferred_element_type=jnp.float32)


def _k3_body(a_ref, h2_ref, as2_ref, an2_ref, b2_ref, out_ref, *, bn):
    i = pl.program_id(0)
    h2f = h2_ref[...]
    h2b = h2_ref[pl.ds(i * bn, bn), :]
    s2 = jax.lax.dot_general(h2b, as2_ref[...], (((1,), (1,)), ((), ())),
                             preferred_element_type=jnp.float32)
    t2 = jax.lax.dot_general(an2_ref[...], h2f, (((1,), (1,)), ((), ())),
                             preferred_element_type=jnp.float32)
    abias = jnp.where(a_ref[...] > 0.0, 0.0, jnp.float32(-1e9))
    logit = _leaky_relu(s2 + t2) + abias
    rmax = jnp.max(logit, axis=1, keepdims=True)
    p = jnp.exp(logit - rmax)
    denom = jnp.sum(p, axis=1, keepdims=True)
    h2agg = jnp.dot(p, h2f, preferred_element_type=jnp.float32) / denom
    h2agg = h2agg + b2_ref[...]
    cmax = jnp.max(h2agg, axis=1, keepdims=True)
    e = jnp.exp(h2agg - cmax)
    out_ref[...] = e / jnp.sum(e, axis=1, keepdims=True)


@jax.jit
def kernel(x, a, layout, kernel1, attn_s1, attn_n1, bias1, Wl, bl,
           kernel2, attn_s2, attn_n2, bias2):
    N, F = x.shape
    H, C = attn_s1.shape
    NC = attn_s2.shape[1]
    HC = H * C

    k1m = kernel1.reshape(F, HC)
    w2 = kernel2.reshape(HC, NC)
    eye = jnp.eye(H, dtype=x.dtype)
    ast = (eye[:, :, None] * attn_s1[None, :, :]).reshape(H, HC)
    ant = (eye[:, :, None] * attn_n1[None, :, :]).reshape(H, HC)
    DL = layout.shape[1]
    DLP = 8
    lay = jnp.pad(layout, ((0, 0), (0, DLP - DL)))
    wlp = jnp.pad(Wl, ((0, DLP - DL), (0, 0)))

    h1, s1, t1, lemb = pl.pallas_call(
        _k1_body,
        out_shape=(
            jax.ShapeDtypeStruct((N, HC), jnp.float32),
            jax.ShapeDtypeStruct((N, H), jnp.float32),
            jax.ShapeDtypeStruct((H, N), jnp.float32),
            jax.ShapeDtypeStruct((N, HC), jnp.float32),
        ),
    )(x, k1m, ast, ant, lay, wlp, bl.reshape(1, HC))

    nblk = N // BN
    h2 = pl.pallas_call(
        functools.partial(_k2_body, heads=H, chan=C),
        grid=(nblk,),
        in_specs=[
            pl.BlockSpec((BN, N), lambda i: (i, 0)),
            pl.BlockSpec((BN, H), lambda i: (i, 0)),
            pl.BlockSpec((H, N), lambda i: (0, 0)),
            pl.BlockSpec((N, HC), lambda i: (0, 0)),
            pl.BlockSpec((BN, HC), lambda i: (i, 0)),
            pl.BlockSpec((1, HC), lambda i: (0, 0)),
            pl.BlockSpec((HC, NC), lambda i: (0, 0)),
        ],
        out_specs=pl.BlockSpec((BN, NC), lambda i: (i, 0)),
        out_shape=jax.ShapeDtypeStruct((N, NC), jnp.float32),
    )(a, s1, t1, h1, lemb, bias1.reshape(1, HC), w2)

    out = pl.pallas_call(
        functools.partial(_k3_body, bn=BN),
        grid=(nblk,),
        in_specs=[
            pl.BlockSpec((BN, N), lambda i: (i, 0)),
            pl.BlockSpec((N, NC), lambda i: (0, 0)),
            pl.BlockSpec((1, NC), lambda i: (0, 0)),
            pl.BlockSpec((1, NC), lambda i: (0, 0)),
            pl.BlockSpec((1, NC), lambda i: (0, 0)),
        ],
        out_specs=pl.BlockSpec((BN, NC), lambda i: (i, 0)),
        out_shape=jax.ShapeDtypeStruct((N, NC), jnp.float32),
    )(a, h2, attn_s2, attn_n2, bias2.reshape(1, NC))
    return out


# speedup vs baseline: 1.4915x; 1.0775x over previous
import functools

import jax
import jax.numpy as jnp
from jax.experimental import pallas as pl

BN = 256


def _leaky_relu(v):
    return jnp.where(v >= 0, v, 0.2 * v)


def _k1_body(x_ref, k1_ref, ast_ref, ant_ref, lay_ref, wl_ref, bl_ref,
             h1_ref, s1_ref, t1_ref, u1_ref, lemb_ref):
    h1 = jnp.dot(x_ref[...], k1_ref[...], preferred_element_type=jnp.float32)
    h1_ref[...] = h1
    s1 = jax.lax.dot_general(
        h1, ast_ref[...], (((1,), (1,)), ((), ())),
        preferred_element_type=jnp.float32)
    s1_ref[...] = s1
    t1 = jax.lax.dot_general(
        ant_ref[...], h1, (((1,), (1,)), ((), ())),
        preferred_element_type=jnp.float32)
    t1_ref[...] = t1
    tmax = jnp.max(t1, axis=1, keepdims=True)
    z = s1 + tmax.T
    u1_ref[...] = jnp.maximum(z, 0.2 * z)
    lemb = jnp.dot(lay_ref[...], wl_ref[...],
                   preferred_element_type=jnp.float32) + bl_ref[...]
    lemb_ref[...] = jnp.maximum(lemb, 0.0)


def _k2_body(a_ref, s1_ref, t1_ref, u1_ref, h1_ref, lemb_ref, b1_ref, w2_ref,
             h2_ref, *, heads, chan):
    ab = a_ref[...]
    s1b = s1_ref[...]
    t1r = t1_ref[...]
    u1b = u1_ref[...]
    h1f = h1_ref[...]
    outs = []
    for h in range(heads):
        z = s1b[:, h:h + 1] + t1r[h:h + 1, :]
        zl = jnp.maximum(z, 0.2 * z)
        p = jnp.exp(zl - u1b[:, h:h + 1]) * ab
        denom = jnp.sum(p, axis=1, keepdims=True)
        outs.append(jnp.dot(p, h1f[:, h * chan:(h + 1) * chan],
                            preferred_element_type=jnp.float32) / denom)
    x1 = jnp.concatenate(outs, axis=1) + b1_ref[...]
    x1 = jnp.where(x1 > 0, x1, jnp.exp(x1) - 1.0)
    x1g = x1 + lemb_ref[...]
    h2_ref[...] = jnp.dot(x1g, w2_ref[...], preferred_element_type=jnp.float32)


def _k3_body(a_ref, h2_ref, as2_ref, an2_ref, b2_ref, out_ref, *, bn):
    i = pl.program_id(0)
    h2f = h2_ref[...]
    h2b = h2_ref[pl.ds(i * bn, bn), :]
    s2 = jax.lax.dot_general(h2b, as2_ref[...], (((1,), (1,)), ((), ())),
                             preferred_element_type=jnp.float32)
    t2 = jax.lax.dot_general(an2_ref[...], h2f, (((1,), (1,)), ((), ())),
                             preferred_element_type=jnp.float32)
    zu = s2 + jnp.max(t2)
    u2 = jnp.maximum(zu, 0.2 * zu)
    z = s2 + t2
    zl = jnp.maximum(z, 0.2 * z)
    p = jnp.exp(zl - u2) * a_ref[...]
    denom = jnp.sum(p, axis=1, keepdims=True)
    h2agg = jnp.dot(p, h2f, preferred_element_type=jnp.float32) / denom
    h2agg = h2agg + b2_ref[...]
    cmax = jnp.max(h2agg, axis=1, keepdims=True)
    e = jnp.exp(h2agg - cmax)
    out_ref[...] = e / jnp.sum(e, axis=1, keepdims=True)


@jax.jit
def kernel(x, a, layout, kernel1, attn_s1, attn_n1, bias1, Wl, bl,
           kernel2, attn_s2, attn_n2, bias2):
    N, F = x.shape
    H, C = attn_s1.shape
    NC = attn_s2.shape[1]
    HC = H * C

    k1m = kernel1.reshape(F, HC)
    w2 = kernel2.reshape(HC, NC)
    eye = jnp.eye(H, dtype=x.dtype)
    ast = (eye[:, :, None] * attn_s1[None, :, :]).reshape(H, HC)
    ant = (eye[:, :, None] * attn_n1[None, :, :]).reshape(H, HC)
    DL = layout.shape[1]
    DLP = 8
    lay = jnp.pad(layout, ((0, 0), (0, DLP - DL)))
    wlp = jnp.pad(Wl, ((0, DLP - DL), (0, 0)))

    h1, s1, t1, u1, lemb = pl.pallas_call(
        _k1_body,
        out_shape=(
            jax.ShapeDtypeStruct((N, HC), jnp.float32),
            jax.ShapeDtypeStruct((N, H), jnp.float32),
            jax.ShapeDtypeStruct((H, N), jnp.float32),
            jax.ShapeDtypeStruct((N, H), jnp.float32),
            jax.ShapeDtypeStruct((N, HC), jnp.float32),
        ),
    )(x, k1m, ast, ant, lay, wlp, bl.reshape(1, HC))

    nblk = N // BN
    h2 = pl.pallas_call(
        functools.partial(_k2_body, heads=H, chan=C),
        grid=(nblk,),
        in_specs=[
            pl.BlockSpec((BN, N), lambda i: (i, 0)),
            pl.BlockSpec((BN, H), lambda i: (i, 0)),
            pl.BlockSpec((H, N), lambda i: (0, 0)),
            pl.BlockSpec((BN, H), lambda i: (i, 0)),
            pl.BlockSpec((N, HC), lambda i: (0, 0)),
            pl.BlockSpec((BN, HC), lambda i: (i, 0)),
            pl.BlockSpec((1, HC), lambda i: (0, 0)),
            pl.BlockSpec((HC, NC), lambda i: (0, 0)),
        ],
        out_specs=pl.BlockSpec((BN, NC), lambda i: (i, 0)),
        out_shape=jax.ShapeDtypeStruct((N, NC), jnp.float32),
    )(a, s1, t1, u1, h1, lemb, bias1.reshape(1, HC), w2)

    out = pl.pallas_call(
        functools.partial(_k3_body, bn=BN),
        grid=(nblk,),
        in_specs=[
            pl.BlockSpec((BN, N), lambda i: (i, 0)),
            pl.BlockSpec((N, NC), lambda i: (0, 0)),
            pl.BlockSpec((1, NC), lambda i: (0, 0)),
            pl.BlockSpec((1, NC), lambda i: (0, 0)),
            pl.BlockSpec((1, NC), lambda i: (0, 0)),
        ],
        out_specs=pl.BlockSpec((BN, NC), lambda i: (i, 0)),
        out_shape=jax.ShapeDtypeStruct((N, NC), jnp.float32),
    )(a, h2, attn_s2, attn_n2, bias2.reshape(1, NC))
    return out


# speedup vs baseline: 1.5905x; 1.0664x over previous
import functools

import jax
import jax.numpy as jnp
from jax.experimental import pallas as pl

BN = 512


def _leaky_relu(v):
    return jnp.where(v >= 0, v, 0.2 * v)


def _k1_body(x_ref, k1_ref, ast_ref, ant_ref, lay_ref, wl_ref, bl_ref,
             h1_ref, s1_ref, t1_ref, u1_ref, lemb_ref):
    h1 = jnp.dot(x_ref[...], k1_ref[...], preferred_element_type=jnp.float32)
    h1_ref[...] = h1
    s1 = jax.lax.dot_general(
        h1, ast_ref[...], (((1,), (1,)), ((), ())),
        preferred_element_type=jnp.float32)
    s1_ref[...] = s1
    t1 = jax.lax.dot_general(
        ant_ref[...], h1, (((1,), (1,)), ((), ())),
        preferred_element_type=jnp.float32)
    t1_ref[...] = t1
    tmax = jnp.max(t1, axis=1, keepdims=True)
    z = s1 + tmax.T
    u1_ref[...] = jnp.maximum(z, 0.2 * z)
    lemb = jnp.dot(lay_ref[...], wl_ref[...],
                   preferred_element_type=jnp.float32) + bl_ref[...]
    lemb_ref[...] = jnp.maximum(lemb, 0.0)


def _k2_body(a_ref, s1_ref, t1_ref, u1_ref, h1_ref, lemb_ref, b1_ref, w2_ref,
             h2_ref, *, heads, chan):
    ab = a_ref[...]
    s1b = s1_ref[...]
    t1r = t1_ref[...]
    u1b = u1_ref[...]
    h1f = h1_ref[...]
    outs = []
    for h in range(heads):
        z = s1b[:, h:h + 1] + t1r[h:h + 1, :]
        zl = jnp.maximum(z, 0.2 * z)
        p = jnp.exp(zl - u1b[:, h:h + 1]) * ab
        denom = jnp.sum(p, axis=1, keepdims=True)
        outs.append(jnp.dot(p, h1f[:, h * chan:(h + 1) * chan],
                            preferred_element_type=jnp.float32) / denom)
    x1 = jnp.concatenate(outs, axis=1) + b1_ref[...]
    x1 = jnp.where(x1 > 0, x1, jnp.exp(x1) - 1.0)
    x1g = x1 + lemb_ref[...]
    h2_ref[...] = jnp.dot(x1g, w2_ref[...], preferred_element_type=jnp.float32)


def _k3_body(a_ref, h2_ref, as2_ref, an2_ref, b2_ref, out_ref, *, bn):
    i = pl.program_id(0)
    h2f = h2_ref[...]
    h2b = h2_ref[pl.ds(i * bn, bn), :]
    s2 = jax.lax.dot_general(h2b, as2_ref[...], (((1,), (1,)), ((), ())),
                             preferred_element_type=jnp.float32)
    t2 = jax.lax.dot_general(an2_ref[...], h2f, (((1,), (1,)), ((), ())),
                             preferred_element_type=jnp.float32)
    zu = s2 + jnp.max(t2)
    u2 = jnp.maximum(zu, 0.2 * zu)
    z = s2 + t2
    zl = jnp.maximum(z, 0.2 * z)
    p = jnp.exp(zl - u2) * a_ref[...]
    denom = jnp.sum(p, axis=1, keepdims=True)
    h2agg = jnp.dot(p, h2f, preferred_element_type=jnp.float32) / denom
    h2agg = h2agg + b2_ref[...]
    cmax = jnp.max(h2agg, axis=1, keepdims=True)
    e = jnp.exp(h2agg - cmax)
    out_ref[...] = e / jnp.sum(e, axis=1, keepdims=True)


@jax.jit
def kernel(x, a, layout, kernel1, attn_s1, attn_n1, bias1, Wl, bl,
           kernel2, attn_s2, attn_n2, bias2):
    N, F = x.shape
    H, C = attn_s1.shape
    NC = attn_s2.shape[1]
    HC = H * C

    k1m = kernel1.reshape(F, HC)
    w2 = kernel2.reshape(HC, NC)
    eye = jnp.eye(H, dtype=x.dtype)
    ast = (eye[:, :, None] * attn_s1[None, :, :]).reshape(H, HC)
    ant = (eye[:, :, None] * attn_n1[None, :, :]).reshape(H, HC)
    DL = layout.shape[1]
    DLP = 8
    lay = jnp.pad(layout, ((0, 0), (0, DLP - DL)))
    wlp = jnp.pad(Wl, ((0, DLP - DL), (0, 0)))

    h1, s1, t1, u1, lemb = pl.pallas_call(
        _k1_body,
        out_shape=(
            jax.ShapeDtypeStruct((N, HC), jnp.float32),
            jax.ShapeDtypeStruct((N, H), jnp.float32),
            jax.ShapeDtypeStruct((H, N), jnp.float32),
            jax.ShapeDtypeStruct((N, H), jnp.float32),
            jax.ShapeDtypeStruct((N, HC), jnp.float32),
        ),
    )(x, k1m, ast, ant, lay, wlp, bl.reshape(1, HC))

    nblk = N // BN
    h2 = pl.pallas_call(
        functools.partial(_k2_body, heads=H, chan=C),
        grid=(nblk,),
        in_specs=[
            pl.BlockSpec((BN, N), lambda i: (i, 0)),
            pl.BlockSpec((BN, H), lambda i: (i, 0)),
            pl.BlockSpec((H, N), lambda i: (0, 0)),
            pl.BlockSpec((BN, H), lambda i: (i, 0)),
            pl.BlockSpec((N, HC), lambda i: (0, 0)),
            pl.BlockSpec((BN, HC), lambda i: (i, 0)),
            pl.BlockSpec((1, HC), lambda i: (0, 0)),
            pl.BlockSpec((HC, NC), lambda i: (0, 0)),
        ],
        out_specs=pl.BlockSpec((BN, NC), lambda i: (i, 0)),
        out_shape=jax.ShapeDtypeStruct((N, NC), jnp.float32),
    )(a, s1, t1, u1, h1, lemb, bias1.reshape(1, HC), w2)

    out = pl.pallas_call(
        functools.partial(_k3_body, bn=BN),
        grid=(nblk,),
        in_specs=[
            pl.BlockSpec((BN, N), lambda i: (i, 0)),
            pl.BlockSpec((N, NC), lambda i: (0, 0)),
            pl.BlockSpec((1, NC), lambda i: (0, 0)),
            pl.BlockSpec((1, NC), lambda i: (0, 0)),
            pl.BlockSpec((1, NC), lambda i: (0, 0)),
        ],
        out_specs=pl.BlockSpec((BN, NC), lambda i: (i, 0)),
        out_shape=jax.ShapeDtypeStruct((N, NC), jnp.float32),
    )(a, h2, attn_s2, attn_n2, bias2.reshape(1, NC))
    return out


# speedup vs baseline: 1.6949x; 1.0656x over previous
import functools

import jax
import jax.numpy as jnp
from jax.experimental import pallas as pl
from jax.experimental.pallas import tpu as pltpu

BN = 512


def _k1_body(x_ref, k1_ref, ast_ref, ant_ref, lay_ref, wl_ref, bl_ref,
             h1_ref, s1_ref, t1_ref, u1_ref, lemb_ref):
    h1 = jnp.dot(x_ref[...], k1_ref[...], preferred_element_type=jnp.float32)
    h1_ref[...] = h1
    s1 = jax.lax.dot_general(
        h1, ast_ref[...], (((1,), (1,)), ((), ())),
        preferred_element_type=jnp.float32)
    s1_ref[...] = s1
    t1 = jax.lax.dot_general(
        ant_ref[...], h1, (((1,), (1,)), ((), ())),
        preferred_element_type=jnp.float32)
    t1_ref[...] = t1
    tmax = jnp.max(t1, axis=1, keepdims=True)
    z = s1 + tmax.T
    u1_ref[...] = jnp.maximum(z, 0.2 * z)
    lemb = jnp.dot(lay_ref[...], wl_ref[...],
                   preferred_element_type=jnp.float32) + bl_ref[...]
    lemb_ref[...] = jnp.maximum(lemb, 0.0)


def _kf_body(a_ref, s1_ref, t1_ref, u1_ref, h1_ref, lemb_ref, b1_ref, w2_ref,
             as2_ref, an2_ref, b2_ref, out_ref, abf_ref, h2_ref,
             *, heads, chan, bn, nblk):
    i = pl.program_id(0)

    def phase_a():
        ab = a_ref[...]
        abf_ref[pl.ds((i % nblk) * bn, bn), :] = ab.astype(jnp.bfloat16)
        s1b = s1_ref[...]
        t1r = t1_ref[...]
        u1b = u1_ref[...]
        h1f = h1_ref[...]
        outs = []
        for h in range(heads):
            z = s1b[:, h:h + 1] + t1r[h:h + 1, :]
            zl = jnp.maximum(z, 0.2 * z)
            p = jnp.exp(zl - u1b[:, h:h + 1]) * ab
            denom = jnp.sum(p, axis=1, keepdims=True)
            outs.append(jnp.dot(p, h1f[:, h * chan:(h + 1) * chan],
                                preferred_element_type=jnp.float32) / denom)
        x1 = jnp.concatenate(outs, axis=1) + b1_ref[...]
        x1 = jnp.where(x1 > 0, x1, jnp.exp(x1) - 1.0)
        x1g = x1 + lemb_ref[...]
        h2_ref[pl.ds((i % nblk) * bn, bn), :] = jnp.dot(
            x1g, w2_ref[...], preferred_element_type=jnp.float32)

    def phase_b():
        j = i - nblk
        h2f = h2_ref[...]
        h2b = h2_ref[pl.ds(j * bn, bn), :]
        s2 = jax.lax.dot_general(h2b, as2_ref[...], (((1,), (1,)), ((), ())),
                                 preferred_element_type=jnp.float32)
        t2 = jax.lax.dot_general(an2_ref[...], h2f, (((1,), (1,)), ((), ())),
                                 preferred_element_type=jnp.float32)
        zu = s2 + jnp.max(t2)
        u2 = jnp.maximum(zu, 0.2 * zu)
        z = s2 + t2
        zl = jnp.maximum(z, 0.2 * z)
        ab = abf_ref[pl.ds(j * bn, bn), :].astype(jnp.float32)
        p = jnp.exp(zl - u2) * ab
        denom = jnp.sum(p, axis=1, keepdims=True)
        h2agg = jnp.dot(p, h2f, preferred_element_type=jnp.float32) / denom
        h2agg = h2agg + b2_ref[...]
        cmax = jnp.max(h2agg, axis=1, keepdims=True)
        e = jnp.exp(h2agg - cmax)
        out_ref[...] = e / jnp.sum(e, axis=1, keepdims=True)

    jax.lax.cond(i < nblk, phase_a, phase_b)


@jax.jit
def kernel(x, a, layout, kernel1, attn_s1, attn_n1, bias1, Wl, bl,
           kernel2, attn_s2, attn_n2, bias2):
    N, F = x.shape
    H, C = attn_s1.shape
    NC = attn_s2.shape[1]
    HC = H * C

    k1m = kernel1.reshape(F, HC)
    w2 = kernel2.reshape(HC, NC)
    eye = jnp.eye(H, dtype=x.dtype)
    ast = (eye[:, :, None] * attn_s1[None, :, :]).reshape(H, HC)
    ant = (eye[:, :, None] * attn_n1[None, :, :]).reshape(H, HC)
    DL = layout.shape[1]
    DLP = 8
    lay = jnp.pad(layout, ((0, 0), (0, DLP - DL)))
    wlp = jnp.pad(Wl, ((0, DLP - DL), (0, 0)))

    h1, s1, t1, u1, lemb = pl.pallas_call(
        _k1_body,
        out_shape=(
            jax.ShapeDtypeStruct((N, HC), jnp.float32),
            jax.ShapeDtypeStruct((N, H), jnp.float32),
            jax.ShapeDtypeStruct((H, N), jnp.float32),
            jax.ShapeDtypeStruct((N, H), jnp.float32),
            jax.ShapeDtypeStruct((N, HC), jnp.float32),
        ),
    )(x, k1m, ast, ant, lay, wlp, bl.reshape(1, HC))

    nblk = N // BN
    out = pl.pallas_call(
        functools.partial(_kf_body, heads=H, chan=C, bn=BN, nblk=nblk),
        grid=(2 * nblk,),
        in_specs=[
            pl.BlockSpec((BN, N), lambda i: (jnp.minimum(i, nblk - 1), 0)),
            pl.BlockSpec((BN, H), lambda i: (i % nblk, 0)),
            pl.BlockSpec((H, N), lambda i: (0, 0)),
            pl.BlockSpec((BN, H), lambda i: (i % nblk, 0)),
            pl.BlockSpec((N, HC), lambda i: (0, 0)),
            pl.BlockSpec((BN, HC), lambda i: (i % nblk, 0)),
            pl.BlockSpec((1, HC), lambda i: (0, 0)),
            pl.BlockSpec((HC, NC), lambda i: (0, 0)),
            pl.BlockSpec((1, NC), lambda i: (0, 0)),
            pl.BlockSpec((1, NC), lambda i: (0, 0)),
            pl.BlockSpec((1, NC), lambda i: (0, 0)),
        ],
        out_specs=pl.BlockSpec((BN, NC), lambda i: (jnp.maximum(i - nblk, 0), 0)),
        out_shape=jax.ShapeDtypeStruct((N, NC), jnp.float32),
        scratch_shapes=[
            pltpu.VMEM((N, N), jnp.bfloat16),
            pltpu.VMEM((N, NC), jnp.float32),
        ],
    )(a, s1, t1, u1, h1, lemb, bias1.reshape(1, HC), w2,
      attn_s2, attn_n2, bias2.reshape(1, NC))
    return out


# speedup vs baseline: 1.8728x; 1.1050x over previous
import functools

import jax
import jax.numpy as jnp
from jax.experimental import pallas as pl
from jax.experimental.pallas import tpu as pltpu

BN = 512
CA = 24
NCA = 1024


def _k1_body(x_ref, k1_ref, ast_ref, ant_ref, lay_ref, wl_ref, bl_ref,
             haug_ref, sa_ref, sb_ref, t1_ref, t1b_ref, lemb_ref,
             *, heads, chan):
    h1 = jnp.dot(x_ref[...], k1_ref[...], preferred_element_type=jnp.float32)
    n = h1.shape[0]
    ones = jnp.ones((n, 1), jnp.float32)
    zeros = jnp.zeros((n, CA - chan - 1), jnp.float32)
    pieces = []
    for h in range(heads):
        pieces += [h1[:, h * chan:(h + 1) * chan], ones, zeros]
    haug_ref[...] = jnp.concatenate(pieces, axis=1)
    s1 = jax.lax.dot_general(
        h1, ast_ref[...], (((1,), (1,)), ((), ())),
        preferred_element_type=jnp.float32)
    t1 = jax.lax.dot_general(
        ant_ref[...], h1, (((1,), (1,)), ((), ())),
        preferred_element_type=jnp.float32)
    t1_ref[...] = t1
    t1b_ref[...] = 0.2 * t1
    tmax = jnp.max(t1, axis=1, keepdims=True)
    z = s1 + tmax.T
    u1 = jnp.maximum(z, 0.2 * z)
    sa_ref[...] = s1 - u1
    sb_ref[...] = 0.2 * s1 - u1
    lemb = jnp.dot(lay_ref[...], wl_ref[...],
                   preferred_element_type=jnp.float32) + bl_ref[...]
    lemb_ref[...] = jnp.maximum(lemb, 0.0)


def _kf_body(a_ref, sa_ref, sb_ref, t1_ref, t1b_ref, haug_ref, lemb_ref,
             b1_ref, w2_ref, as2_ref, an2_ref, b2_ref, out_ref,
             abf_ref, h2_ref, *, heads, chan, nc, bn, nblk):
    i = pl.program_id(0)

    def phase_a():
        ab = a_ref[...]
        abf_ref[pl.ds((i % nblk) * bn, bn), :] = ab.astype(jnp.bfloat16)
        sab = sa_ref[...]
        sbb = sb_ref[...]
        t1r = t1_ref[...]
        t1br = t1b_ref[...]
        haug = haug_ref[...]
        outs = []
        for h in range(heads):
            zl = jnp.maximum(sab[:, h:h + 1] + t1r[h:h + 1, :],
                             sbb[:, h:h + 1] + t1br[h:h + 1, :])
            p = jnp.exp(zl) * ab
            aug = jnp.dot(p, haug[:, h * CA:(h + 1) * CA],
                          preferred_element_type=jnp.float32)
            outs.append(aug[:, :chan] / aug[:, chan:chan + 1])
        x1 = jnp.concatenate(outs, axis=1) + b1_ref[...]
        x1 = jnp.where(x1 > 0, x1, jnp.exp(x1) - 1.0)
        x1g = x1 + lemb_ref[...]
        h2b = jnp.dot(x1g, w2_ref[...], preferred_element_type=jnp.float32)
        ones = jnp.ones((bn, 1), jnp.float32)
        zeros = jnp.zeros((bn, NCA - nc - 1), jnp.float32)
        h2_ref[pl.ds((i % nblk) * bn, bn), :] = jnp.concatenate(
            [h2b, ones, zeros], axis=1)

    def phase_b():
        j = i - nblk
        h2f = h2_ref[...]
        h2b = h2_ref[pl.ds(j * bn, bn), :]
        s2 = jax.lax.dot_general(h2b, as2_ref[...], (((1,), (1,)), ((), ())),
                                 preferred_element_type=jnp.float32)
        t2 = jax.lax.dot_general(an2_ref[...], h2f, (((1,), (1,)), ((), ())),
                                 preferred_element_type=jnp.float32)
        zu = s2 + jnp.max(t2)
        u2 = jnp.maximum(zu, 0.2 * zu)
        sa2 = s2 - u2
        sb2 = 0.2 * s2 - u2
        zl = jnp.maximum(sa2 + t2, sb2 + 0.2 * t2)
        ab = abf_ref[pl.ds(j * bn, bn), :].astype(jnp.float32)
        p = jnp.exp(zl) * ab
        agg = jnp.dot(p, h2f, preferred_element_type=jnp.float32)
        h2agg = agg[:, :nc] / agg[:, nc:nc + 1] + b2_ref[...]
        cmax = jnp.max(h2agg, axis=1, keepdims=True)
        e = jnp.exp(h2agg - cmax)
        out_ref[...] = e / jnp.sum(e, axis=1, keepdims=True)

    jax.lax.cond(i < nblk, phase_a, phase_b)


@jax.jit
def kernel(x, a, layout, kernel1, attn_s1, attn_n1, bias1, Wl, bl,
           kernel2, attn_s2, attn_n2, bias2):
    N, F = x.shape
    H, C = attn_s1.shape
    NC = attn_s2.shape[1]
    HC = H * C

    k1m = kernel1.reshape(F, HC)
    w2 = kernel2.reshape(HC, NC)
    eye = jnp.eye(H, dtype=x.dtype)
    ast = (eye[:, :, None] * attn_s1[None, :, :]).reshape(H, HC)
    ant = (eye[:, :, None] * attn_n1[None, :, :]).reshape(H, HC)
    DL = layout.shape[1]
    DLP = 8
    lay = jnp.pad(layout, ((0, 0), (0, DLP - DL)))
    wlp = jnp.pad(Wl, ((0, DLP - DL), (0, 0)))
    as2p = jnp.pad(attn_s2, ((0, 0), (0, NCA - NC)))
    an2p = jnp.pad(attn_n2, ((0, 0), (0, NCA - NC)))

    haug, sa, sb, t1, t1b, lemb = pl.pallas_call(
        functools.partial(_k1_body, heads=H, chan=C),
        out_shape=(
            jax.ShapeDtypeStruct((N, H * CA), jnp.float32),
            jax.ShapeDtypeStruct((N, H), jnp.float32),
            jax.ShapeDtypeStruct((N, H), jnp.float32),
            jax.ShapeDtypeStruct((H, N), jnp.float32),
            jax.ShapeDtypeStruct((H, N), jnp.float32),
            jax.ShapeDtypeStruct((N, HC), jnp.float32),
        ),
    )(x, k1m, ast, ant, lay, wlp, bl.reshape(1, HC))

    nblk = N // BN
    out = pl.pallas_call(
        functools.partial(_kf_body, heads=H, chan=C, nc=NC, bn=BN, nblk=nblk),
        grid=(2 * nblk,),
        in_specs=[
            pl.BlockSpec((BN, N), lambda i: (jnp.minimum(i, nblk - 1), 0)),
            pl.BlockSpec((BN, H), lambda i: (i % nblk, 0)),
            pl.BlockSpec((BN, H), lambda i: (i % nblk, 0)),
            pl.BlockSpec((H, N), lambda i: (0, 0)),
            pl.BlockSpec((H, N), lambda i: (0, 0)),
            pl.BlockSpec((N, H * CA), lambda i: (0, 0)),
            pl.BlockSpec((BN, HC), lambda i: (i % nblk, 0)),
            pl.BlockSpec((1, HC), lambda i: (0, 0)),
            pl.BlockSpec((HC, NC), lambda i: (0, 0)),
            pl.BlockSpec((1, NCA), lambda i: (0, 0)),
            pl.BlockSpec((1, NCA), lambda i: (0, 0)),
            pl.BlockSpec((1, NC), lambda i: (0, 0)),
        ],
        out_specs=pl.BlockSpec((BN, NC), lambda i: (jnp.maximum(i - nblk, 0), 0)),
        out_shape=jax.ShapeDtypeStruct((N, NC), jnp.float32),
        scratch_shapes=[
            pltpu.VMEM((N, N), jnp.bfloat16),
            pltpu.VMEM((N, NCA), jnp.float32),
        ],
    )(a, sa, sb, t1, t1b, haug, lemb, bias1.reshape(1, HC), w2,
      as2p, an2p, bias2.reshape(1, NC))
    return out


# speedup vs baseline: 2.2385x; 1.1953x over previous
import functools

import jax
import jax.numpy as jnp
from jax.experimental import pallas as pl
from jax.experimental.pallas import tpu as pltpu

BN = 512
CA = 24
NCA = 1024


def _body(x_ref, a_ref, lay_ref, k1_ref, as1_ref, an1_ref, b1_ref,
          wl_ref, bl_ref, w2_ref, as2_ref, an2_ref, b2_ref,
          out_ref,
          abf_ref, h2_ref, haug_ref, sa_ref, sb_ref, t1_ref, t1b_ref,
          lemb_ref,
          *, heads, chan, nc, dl, bn, nblk):
    i = pl.program_id(0)

    def setup():
        n = x_ref.shape[0]
        h1 = jnp.dot(x_ref[...], k1_ref[...],
                     preferred_element_type=jnp.float32)
        ones = jnp.ones((n, 1), jnp.float32)
        zeros = jnp.zeros((n, CA - chan - 1), jnp.float32)
        pieces = []
        for h in range(heads):
            pieces += [h1[:, h * chan:(h + 1) * chan], ones, zeros]
        haug_ref[...] = jnp.concatenate(pieces, axis=1)
        hc = heads * chan
        tile_s = jnp.concatenate([as1_ref[...]] * heads, axis=1)
        tile_n = jnp.concatenate([an1_ref[...]] * heads, axis=1)
        row = jax.lax.broadcasted_iota(jnp.int32, (heads, hc), 0)
        grp = jax.lax.broadcasted_iota(jnp.int32, (heads, hc), 1) // chan
        blkdiag = row == grp
        ast = jnp.where(blkdiag, tile_s, 0.0)
        ant = jnp.where(blkdiag, tile_n, 0.0)
        s1 = jax.lax.dot_general(h1, ast, (((1,), (1,)), ((), ())),
                                 preferred_element_type=jnp.float32)
        t1 = jax.lax.dot_general(ant, h1, (((1,), (1,)), ((), ())),
                                 preferred_element_type=jnp.float32)
        t1_ref[...] = t1
        t1b_ref[...] = 0.2 * t1
        tmax = jnp.max(t1, axis=1, keepdims=True)
        z = s1 + tmax.T
        u1 = jnp.maximum(z, 0.2 * z)
        sa_ref[...] = s1 - u1
        sb_ref[...] = 0.2 * s1 - u1
        lemb = jnp.dot(lay_ref[...], wl_ref[...],
                       preferred_element_type=jnp.float32) + bl_ref[...]
        lemb_ref[...] = jnp.maximum(lemb, 0.0)

    def phase_a():
        j = (i - 1) % nblk
        ab = a_ref[...]
        abf_ref[pl.ds(j * bn, bn), :] = ab.astype(jnp.bfloat16)
        sab = sa_ref[pl.ds(j * bn, bn), :]
        sbb = sb_ref[pl.ds(j * bn, bn), :]
        t1r = t1_ref[...]
        t1br = t1b_ref[...]
        haug = haug_ref[...]
        outs = []
        for h in range(heads):
            zl = jnp.maximum(sab[:, h:h + 1] + t1r[h:h + 1, :],
                             sbb[:, h:h + 1] + t1br[h:h + 1, :])
            p = jnp.exp(zl) * ab
            aug = jnp.dot(p, haug[:, h * CA:(h + 1) * CA],
                          preferred_element_type=jnp.float32)
            outs.append(aug[:, :chan] / aug[:, chan:chan + 1])
        x1 = jnp.concatenate(outs, axis=1) + b1_ref[...]
        x1 = jnp.where(x1 > 0, x1, jnp.exp(x1) - 1.0)
        x1g = x1 + lemb_ref[pl.ds(j * bn, bn), :]
        h2b = jnp.dot(x1g, w2_ref[...], preferred_element_type=jnp.float32)
        ones = jnp.ones((bn, 1), jnp.float32)
        zeros = jnp.zeros((bn, NCA - nc - 1), jnp.float32)
        h2_ref[pl.ds(j * bn, bn), :] = jnp.concatenate(
            [h2b, ones, zeros], axis=1)

    def phase_b():
        j = i - nblk - 1
        h2f = h2_ref[...]
        h2b = h2_ref[pl.ds(j * bn, bn), :nc]
        s2 = jax.lax.dot_general(h2b, as2_ref[...], (((1,), (1,)), ((), ())),
                                 preferred_element_type=jnp.float32)
        t2 = jax.lax.dot_general(an2_ref[...], h2f[:, :nc],
                                 (((1,), (1,)), ((), ())),
                                 preferred_element_type=jnp.float32)
        zu = s2 + jnp.max(t2)
        u2 = jnp.maximum(zu, 0.2 * zu)
        sa2 = s2 - u2
        sb2 = 0.2 * s2 - u2
        zl = jnp.maximum(sa2 + t2, sb2 + 0.2 * t2)
        ab = abf_ref[pl.ds(j * bn, bn), :].astype(jnp.float32)
        p = jnp.exp(zl) * ab
        agg = jnp.dot(p, h2f, preferred_element_type=jnp.float32)
        h2agg = agg[:, :nc] / agg[:, nc:nc + 1] + b2_ref[...]
        cmax = jnp.max(h2agg, axis=1, keepdims=True)
        e = jnp.exp(h2agg - cmax)
        out_ref[...] = e / jnp.sum(e, axis=1, keepdims=True)

    idx = (i > 0).astype(jnp.int32) + (i > nblk).astype(jnp.int32)
    jax.lax.switch(idx, [setup, phase_a, phase_b])


@jax.jit
def kernel(x, a, layout, kernel1, attn_s1, attn_n1, bias1, Wl, bl,
           kernel2, attn_s2, attn_n2, bias2):
    N, F = x.shape
    H, C = attn_s1.shape
    NC = attn_s2.shape[1]
    HC = H * C
    DL = layout.shape[1]

    k1m = kernel1.reshape(F, HC)
    w2 = kernel2.reshape(HC, NC)
    nblk = N // BN

    def a_map(i):
        return (jnp.where(i <= nblk, jnp.maximum(i - 1, 0), i - nblk - 1), 0)

    out = pl.pallas_call(
        functools.partial(_body, heads=H, chan=C, nc=NC, dl=DL, bn=BN,
                          nblk=nblk),
        grid=(1 + 2 * nblk,),
        in_specs=[
            pl.BlockSpec((N, F), lambda i: (0, 0)),
            pl.BlockSpec((BN, N), a_map),
            pl.BlockSpec((N, DL), lambda i: (0, 0)),
            pl.BlockSpec((F, HC), lambda i: (0, 0)),
            pl.BlockSpec((H, C), lambda i: (0, 0)),
            pl.BlockSpec((H, C), lambda i: (0, 0)),
            pl.BlockSpec((1, HC), lambda i: (0, 0)),
            pl.BlockSpec((DL, HC), lambda i: (0, 0)),
            pl.BlockSpec((1, HC), lambda i: (0, 0)),
            pl.BlockSpec((HC, NC), lambda i: (0, 0)),
            pl.BlockSpec((1, NC), lambda i: (0, 0)),
            pl.BlockSpec((1, NC), lambda i: (0, 0)),
            pl.BlockSpec((1, NC), lambda i: (0, 0)),
        ],
        out_specs=pl.BlockSpec(
            (BN, NC), lambda i: (jnp.maximum(i - nblk - 1, 0), 0)),
        out_shape=jax.ShapeDtypeStruct((N, NC), jnp.float32),
        scratch_shapes=[
            pltpu.VMEM((N, N), jnp.bfloat16),
            pltpu.VMEM((N, NCA), jnp.float32),
            pltpu.VMEM((N, H * CA), jnp.float32),
            pltpu.VMEM((N, H), jnp.float32),
            pltpu.VMEM((N, H), jnp.float32),
            pltpu.VMEM((H, N), jnp.float32),
            pltpu.VMEM((H, N), jnp.float32),
            pltpu.VMEM((N, HC), jnp.float32),
        ],
    )(x, a, layout, k1m, attn_s1, attn_n1, bias1.reshape(1, HC),
      Wl, bl.reshape(1, HC), w2, attn_s2, attn_n2, bias2.reshape(1, NC))
    return out


# speedup vs baseline: 2.4114x; 1.0772x over previous
import functools

import jax
import jax.numpy as jnp
from jax.experimental import pallas as pl
from jax.experimental.pallas import tpu as pltpu

BN = 512
CA = 24
NCA = 1024


def _body(x_ref, a_ref, lay_ref, k1_ref, as1_ref, an1_ref, b1_ref,
          wl_ref, bl_ref, w2_ref, as2_ref, an2_ref, b2_ref,
          out_ref,
          abf_ref, h2_ref, haug_ref, sa_ref, sb_ref, t1_ref, t1b_ref,
          lemb_ref, s2_ref, t2_ref,
          *, heads, chan, nc, dl, bn, nblk):
    i = pl.program_id(0)

    def setup():
        n = x_ref.shape[0]
        h1 = jnp.dot(x_ref[...], k1_ref[...],
                     preferred_element_type=jnp.float32)
        ones = jnp.ones((n, 1), jnp.float32)
        zeros = jnp.zeros((n, CA - chan - 1), jnp.float32)
        pieces = []
        for h in range(heads):
            pieces += [h1[:, h * chan:(h + 1) * chan], ones, zeros]
        haug_ref[...] = jnp.concatenate(pieces, axis=1).astype(jnp.bfloat16)
        hc = heads * chan
        tile_s = jnp.concatenate([as1_ref[...]] * heads, axis=1)
        tile_n = jnp.concatenate([an1_ref[...]] * heads, axis=1)
        row = jax.lax.broadcasted_iota(jnp.int32, (heads, hc), 0)
        grp = jax.lax.broadcasted_iota(jnp.int32, (heads, hc), 1) // chan
        blkdiag = row == grp
        ast = jnp.where(blkdiag, tile_s, 0.0)
        ant = jnp.where(blkdiag, tile_n, 0.0)
        s1 = jax.lax.dot_general(h1, ast, (((1,), (1,)), ((), ())),
                                 preferred_element_type=jnp.float32)
        t1 = jax.lax.dot_general(ant, h1, (((1,), (1,)), ((), ())),
                                 preferred_element_type=jnp.float32)
        t1_ref[...] = t1
        t1b_ref[...] = 0.2 * t1
        tmax = jnp.max(t1, axis=1, keepdims=True)
        z = s1 + tmax.T
        u1 = jnp.maximum(z, 0.2 * z)
        sa_ref[...] = s1 - u1
        sb_ref[...] = 0.2 * s1 - u1
        lemb = jnp.dot(lay_ref[...], wl_ref[...],
                       preferred_element_type=jnp.float32) + bl_ref[...]
        lemb_ref[...] = jnp.maximum(lemb, 0.0)

    def phase_a():
        j = (i - 1) % nblk
        ab = a_ref[...]
        abb = ab.astype(jnp.bfloat16)
        abf_ref[pl.ds(j * bn, bn), :] = abb
        sab = sa_ref[pl.ds(j * bn, bn), :]
        sbb = sb_ref[pl.ds(j * bn, bn), :]
        t1r = t1_ref[...]
        t1br = t1b_ref[...]
        haug = haug_ref[...]
        outs = []
        for h in range(heads):
            zl = jnp.maximum(sab[:, h:h + 1] + t1r[h:h + 1, :],
                             sbb[:, h:h + 1] + t1br[h:h + 1, :])
            p = jnp.exp(zl).astype(jnp.bfloat16) * abb
            aug = jnp.dot(p, haug[:, h * CA:(h + 1) * CA],
                          preferred_element_type=jnp.float32)
            outs.append(aug[:, :chan] / aug[:, chan:chan + 1])
        x1 = jnp.concatenate(outs, axis=1) + b1_ref[...]
        x1 = jnp.where(x1 > 0, x1, jnp.exp(x1) - 1.0)
        x1g = x1 + lemb_ref[pl.ds(j * bn, bn), :]
        h2b = jnp.dot(x1g, w2_ref[...], preferred_element_type=jnp.float32)
        ones = jnp.ones((bn, 1), jnp.float32)
        zeros = jnp.zeros((bn, NCA - nc - 1), jnp.float32)
        h2_ref[pl.ds(j * bn, bn), :] = jnp.concatenate(
            [h2b, ones, zeros], axis=1).astype(jnp.bfloat16)
        s2_ref[pl.ds(j * bn, bn), :] = jax.lax.dot_general(
            h2b, as2_ref[...], (((1,), (1,)), ((), ())),
            preferred_element_type=jnp.float32)
        t2_ref[0:1, pl.ds(j * bn, bn)] = jax.lax.dot_general(
            an2_ref[...], h2b, (((1,), (1,)), ((), ())),
            preferred_element_type=jnp.float32)

    def phase_b():
        j = i - nblk - 1
        h2f = h2_ref[...]
        s2 = s2_ref[pl.ds(j * bn, bn), :]
        t2 = t2_ref[...]
        zu = s2 + jnp.max(t2)
        u2 = jnp.maximum(zu, 0.2 * zu)
        sa2 = s2 - u2
        sb2 = 0.2 * s2 - u2
        zl = jnp.maximum(sa2 + t2, sb2 + 0.2 * t2)
        p = jnp.exp(zl).astype(jnp.bfloat16) * abf_ref[pl.ds(j * bn, bn), :]
        agg = jnp.dot(p, h2f, preferred_element_type=jnp.float32)
        h2agg = agg[:, :nc] / agg[:, nc:nc + 1] + b2_ref[...]
        cmax = jnp.max(h2agg, axis=1, keepdims=True)
        e = jnp.exp(h2agg - cmax)
        out_ref[...] = e / jnp.sum(e, axis=1, keepdims=True)

    idx = (i > 0).astype(jnp.int32) + (i > nblk).astype(jnp.int32)
    jax.lax.switch(idx, [setup, phase_a, phase_b])


@jax.jit
def kernel(x, a, layout, kernel1, attn_s1, attn_n1, bias1, Wl, bl,
           kernel2, attn_s2, attn_n2, bias2):
    N, F = x.shape
    H, C = attn_s1.shape
    NC = attn_s2.shape[1]
    HC = H * C
    DL = layout.shape[1]

    k1m = kernel1.reshape(F, HC)
    w2 = kernel2.reshape(HC, NC)
    nblk = N // BN

    def a_map(i):
        return (jnp.where(i <= nblk, jnp.maximum(i - 1, 0), i - nblk - 1), 0)

    out = pl.pallas_call(
        functools.partial(_body, heads=H, chan=C, nc=NC, dl=DL, bn=BN,
                          nblk=nblk),
        grid=(1 + 2 * nblk,),
        in_specs=[
            pl.BlockSpec((N, F), lambda i: (0, 0)),
            pl.BlockSpec((BN, N), a_map),
            pl.BlockSpec((N, DL), lambda i: (0, 0)),
            pl.BlockSpec((F, HC), lambda i: (0, 0)),
            pl.BlockSpec((H, C), lambda i: (0, 0)),
            pl.BlockSpec((H, C), lambda i: (0, 0)),
            pl.BlockSpec((1, HC), lambda i: (0, 0)),
            pl.BlockSpec((DL, HC), lambda i: (0, 0)),
            pl.BlockSpec((1, HC), lambda i: (0, 0)),
            pl.BlockSpec((HC, NC), lambda i: (0, 0)),
            pl.BlockSpec((1, NC), lambda i: (0, 0)),
            pl.BlockSpec((1, NC), lambda i: (0, 0)),
            pl.BlockSpec((1, NC), lambda i: (0, 0)),
        ],
        out_specs=pl.BlockSpec(
            (BN, NC), lambda i: (jnp.maximum(i - nblk - 1, 0), 0)),
        out_shape=jax.ShapeDtypeStruct((N, NC), jnp.float32),
        scratch_shapes=[
            pltpu.VMEM((N, N), jnp.bfloat16),
            pltpu.VMEM((N, NCA), jnp.bfloat16),
            pltpu.VMEM((N, H * CA), jnp.bfloat16),
            pltpu.VMEM((N, H), jnp.float32),
            pltpu.VMEM((N, H), jnp.float32),
            pltpu.VMEM((H, N), jnp.float32),
            pltpu.VMEM((H, N), jnp.float32),
            pltpu.VMEM((N, HC), jnp.float32),
            pltpu.VMEM((N, 1), jnp.float32),
            pltpu.VMEM((1, N), jnp.float32),
        ],
    )(x, a, layout, k1m, attn_s1, attn_n1, bias1.reshape(1, HC),
      Wl, bl.reshape(1, HC), w2, attn_s2, attn_n2, bias2.reshape(1, NC))
    return out


# speedup vs baseline: 2.5122x; 1.0418x over previous
import functools

import jax
import jax.numpy as jnp
from jax.experimental import pallas as pl
from jax.experimental.pallas import tpu as pltpu

BN = 512
CA = 24
NCA = 1024


def _body(x_ref, a_ref, lay_ref, k1_ref, as1_ref, an1_ref, b1_ref,
          wl_ref, bl_ref, w2_ref, as2_ref, an2_ref, b2_ref,
          out_ref,
          abf_ref, h2_ref, haug_ref, ea_ref, eb_ref, et_ref, ef_ref,
          lemb_ref, s2_ref, t2_ref,
          *, heads, chan, nc, bn, nblk):
    i = pl.program_id(0)

    def setup():
        n = x_ref.shape[0]
        xv = x_ref[...]
        ones = jnp.ones((n, 1), jnp.float32)
        zeros = jnp.zeros((n, CA - chan - 1), jnp.float32)
        pieces = []
        s1s = []
        t1s = []
        for h in range(heads):
            h1h = jnp.dot(xv, k1_ref[:, h, :],
                          preferred_element_type=jnp.float32)
            pieces += [h1h, ones, zeros]
            s1s.append(jax.lax.dot_general(
                h1h, as1_ref[h:h + 1, :], (((1,), (1,)), ((), ())),
                preferred_element_type=jnp.float32))
            t1s.append(jax.lax.dot_general(
                an1_ref[h:h + 1, :], h1h, (((1,), (1,)), ((), ())),
                preferred_element_type=jnp.float32))
        haug_ref[...] = jnp.concatenate(pieces, axis=1).astype(jnp.bfloat16)
        s1 = jnp.concatenate(s1s, axis=1)
        t1 = jnp.concatenate(t1s, axis=0)
        tmax = jnp.max(t1, axis=1, keepdims=True)
        z = s1 + tmax.T
        u1 = jnp.maximum(z, 0.2 * z)
        ea_ref[...] = jnp.exp(s1 - u1 + tmax.T)
        eb_ref[...] = jnp.exp(0.2 * s1 - u1 + 0.2 * tmax.T)
        et_ref[...] = jnp.exp(t1 - tmax)
        ef_ref[...] = jnp.exp(0.2 * (t1 - tmax))
        lemb = jnp.dot(lay_ref[...], wl_ref[...],
                       preferred_element_type=jnp.float32) + bl_ref[...][None, :]
        lemb_ref[...] = jnp.maximum(lemb, 0.0)

    def phase_a():
        j = (i - 1) % nblk
        ab = a_ref[...]
        abb = ab.astype(jnp.bfloat16)
        abf_ref[pl.ds(j * bn, bn), :] = abb
        eab = ea_ref[pl.ds(j * bn, bn), :]
        ebb = eb_ref[pl.ds(j * bn, bn), :]
        etr = et_ref[...]
        efr = ef_ref[...]
        haug = haug_ref[...]
        outs = []
        for h in range(heads):
            pz = jnp.maximum(eab[:, h:h + 1] * etr[h:h + 1, :],
                             ebb[:, h:h + 1] * efr[h:h + 1, :])
            p = pz.astype(jnp.bfloat16) * abb
            aug = jnp.dot(p, haug[:, h * CA:(h + 1) * CA],
                          preferred_element_type=jnp.float32)
            outs.append(aug[:, :chan] / aug[:, chan:chan + 1])
        x1 = jnp.concatenate(outs, axis=1) + b1_ref[...][None, :]
        x1 = jnp.where(x1 > 0, x1, jnp.exp(x1) - 1.0)
        x1g = x1 + lemb_ref[pl.ds(j * bn, bn), :]
        h2b = jnp.dot(x1g, w2_ref[:, 0, :], preferred_element_type=jnp.float32)
        ones = jnp.ones((bn, 1), jnp.float32)
        zeros = jnp.zeros((bn, NCA - nc - 1), jnp.float32)
        h2_ref[pl.ds(j * bn, bn), :] = jnp.concatenate(
            [h2b, ones, zeros], axis=1).astype(jnp.bfloat16)
        s2_ref[pl.ds(j * bn, bn), :] = jax.lax.dot_general(
            h2b, as2_ref[...], (((1,), (1,)), ((), ())),
            preferred_element_type=jnp.float32)
        t2_ref[0:1, pl.ds(j * bn, bn)] = jax.lax.dot_general(
            an2_ref[...], h2b, (((1,), (1,)), ((), ())),
            preferred_element_type=jnp.float32)

    def phase_b():
        j = i - nblk - 1
        h2f = h2_ref[...]
        s2 = s2_ref[pl.ds(j * bn, bn), :]
        t2 = t2_ref[...]
        t2max = jnp.max(t2)
        zu = s2 + t2max
        u2 = jnp.maximum(zu, 0.2 * zu)
        ea2 = jnp.exp(s2 - u2 + t2max)
        eb2 = jnp.exp(0.2 * s2 - u2 + 0.2 * t2max)
        et2 = jnp.exp(t2 - t2max)
        ef2 = jnp.exp(0.2 * (t2 - t2max))
        pz = jnp.maximum(ea2 * et2, eb2 * ef2)
        p = pz.astype(jnp.bfloat16) * abf_ref[pl.ds(j * bn, bn), :]
        agg = jnp.dot(p, h2f, preferred_element_type=jnp.float32)
        h2agg = agg[:, :nc] / agg[:, nc:nc + 1] + b2_ref[...][None, :]
        cmax = jnp.max(h2agg, axis=1, keepdims=True)
        e = jnp.exp(h2agg - cmax)
        out_ref[...] = e / jnp.sum(e, axis=1, keepdims=True)

    idx = (i > 0).astype(jnp.int32) + (i > nblk).astype(jnp.int32)
    jax.lax.switch(idx, [setup, phase_a, phase_b])


@jax.jit
def kernel(x, a, layout, kernel1, attn_s1, attn_n1, bias1, Wl, bl,
           kernel2, attn_s2, attn_n2, bias2):
    N, F = x.shape
    H, C = attn_s1.shape
    NC = attn_s2.shape[1]
    HC = H * C
    DL = layout.shape[1]
    nblk = N // BN

    def a_map(i):
        return (jnp.where(i <= nblk, jnp.maximum(i - 1, 0), i - nblk - 1), 0)

    out = pl.pallas_call(
        functools.partial(_body, heads=H, chan=C, nc=NC, bn=BN, nblk=nblk),
        grid=(1 + 2 * nblk,),
        in_specs=[
            pl.BlockSpec((N, F), lambda i: (0, 0)),
            pl.BlockSpec((BN, N), a_map),
            pl.BlockSpec((N, DL), lambda i: (0, 0)),
            pl.BlockSpec((F, H, C), lambda i: (0, 0, 0)),
            pl.BlockSpec((H, C), lambda i: (0, 0)),
            pl.BlockSpec((H, C), lambda i: (0, 0)),
            pl.BlockSpec((HC,), lambda i: (0,)),
            pl.BlockSpec((DL, HC), lambda i: (0, 0)),
            pl.BlockSpec((HC,), lambda i: (0,)),
            pl.BlockSpec((HC, 1, NC), lambda i: (0, 0, 0)),
            pl.BlockSpec((1, NC), lambda i: (0, 0)),
            pl.BlockSpec((1, NC), lambda i: (0, 0)),
            pl.BlockSpec((NC,), lambda i: (0,)),
        ],
        out_specs=pl.BlockSpec(
            (BN, NC), lambda i: (jnp.maximum(i - nblk - 1, 0), 0)),
        out_shape=jax.ShapeDtypeStruct((N, NC), jnp.float32),
        scratch_shapes=[
            pltpu.VMEM((N, N), jnp.bfloat16),
            pltpu.VMEM((N, NCA), jnp.bfloat16),
            pltpu.VMEM((N, H * CA), jnp.bfloat16),
            pltpu.VMEM((N, H), jnp.float32),
            pltpu.VMEM((N, H), jnp.float32),
            pltpu.VMEM((H, N), jnp.float32),
            pltpu.VMEM((H, N), jnp.float32),
            pltpu.VMEM((N, HC), jnp.float32),
            pltpu.VMEM((N, 1), jnp.float32),
            pltpu.VMEM((1, N), jnp.float32),
        ],
    )(x, a, layout, kernel1, attn_s1, attn_n1, bias1,
      Wl, bl, kernel2, attn_s2, attn_n2, bias2)
    return out
